# baseline (speedup 1.0000x reference)
import jax
import jax.numpy as jnp
from jax import lax
from jax.experimental import pallas as pl
from jax.experimental.pallas import tpu as pltpu
from jax.experimental.pallas import tpu_sc as plsc

D_MODEL = 64
D_VOCAB = 100000
BATCH = 4096
POS = 200

B_TOTAL = BATCH * POS
NW = 32
B_PER_W = B_TOTAL // NW
CHUNK = 128
N_CHUNK = B_PER_W // CHUNK

_TBLK = 4096


def _table_body(w_ref, out_ref):
    t = w_ref[...].T.reshape(_TBLK // 2, 2, D_MODEL)
    out_ref[...] = jnp.concatenate([t[:, 0, :], t[:, 1, :]], axis=1)


def _build_table(W_E):
    return pl.pallas_call(
        _table_body,
        grid=(pl.cdiv(D_VOCAB, _TBLK),),
        in_specs=[pl.BlockSpec((D_MODEL, _TBLK), lambda i: (0, i))],
        out_specs=pl.BlockSpec((_TBLK // 2, 2 * D_MODEL), lambda i: (i, 0)),
        out_shape=jax.ShapeDtypeStruct((D_VOCAB // 2, 2 * D_MODEL), jnp.float32),
    )(W_E)


_RBLK = 4096


def _repack_body(in_ref, out_ref):
    x = in_ref[...]
    pair = jnp.stack([x[:, :D_MODEL], x[:, D_MODEL:]], axis=1)
    out_ref[...] = pair.reshape(2 * _RBLK, D_MODEL)


def _repack(packed):
    return pl.pallas_call(
        _repack_body,
        grid=(B_TOTAL // (2 * _RBLK),),
        in_specs=[pl.BlockSpec((_RBLK, 2 * D_MODEL), lambda i: (i, 0))],
        out_specs=pl.BlockSpec((2 * _RBLK, D_MODEL), lambda i: (i, 0)),
        out_shape=jax.ShapeDtypeStruct((B_TOTAL, D_MODEL), jnp.float32),
    )(packed)


K = 4
NG = N_CHUNK // K


def _gather_body(idx_hbm, table_hbm, out_hbm, idx_v, buf_a, buf_b,
                 gsem_a, gsem_b, wsem_a, wsem_b):
    wid = lax.axis_index("s") * 2 + lax.axis_index("c")
    row_base = wid * N_CHUNK
    out_base = wid * B_PER_W

    pltpu.sync_copy(idx_hbm.at[pl.ds(row_base, N_CHUNK)], idx_v)

    bufs = (buf_a, buf_b)
    gsems = (gsem_a, gsem_b)
    wsems = (wsem_a, wsem_b)

    def fire(g, b):
        for j in range(K):
            pltpu.async_copy(
                table_hbm.at[idx_v.at[g * K + j]],
                bufs[b].at[pl.ds(j * CHUNK, CHUNK)],
                gsems[b],
            )

    def drain(g, b):
        for j in range(K):
            pltpu.make_async_copy(
                table_hbm.at[idx_v.at[g * K + j]],
                bufs[b].at[pl.ds(j * CHUNK, CHUNK)],
                gsems[b],
            ).wait()

    def write(g, b):
        pltpu.async_copy(
            bufs[b],
            out_hbm.at[pl.ds(out_base + g * K * CHUNK, K * CHUNK)],
            wsems[b],
        )

    def wait_write(g, b):
        pltpu.make_async_copy(
            bufs[b],
            out_hbm.at[pl.ds(out_base + g * K * CHUNK, K * CHUNK)],
            wsems[b],
        ).wait()

    fire(0, 0)

    def body(i, carry):
        g0 = 2 * i
        g1 = g0 + 1

        @pl.when(i > 0)
        def _():
            wait_write(g1 - 2, 1)
        fire(g1, 1)

        drain(g0, 0)
        write(g0, 0)

        @pl.when(i < NG // 2 - 1)
        def _():
            wait_write(g0, 0)
            fire(g0 + 2, 0)

        drain(g1, 1)
        write(g1, 1)
        return carry

    lax.fori_loop(0, NG // 2, body, 0)
    wait_write(NG - 2, 0)
    wait_write(NG - 1, 1)


@jax.jit
def _embed(x, W_E):
    table = _build_table(W_E).reshape(D_VOCAB, D_MODEL)
    idx = x.reshape(B_TOTAL // CHUNK, CHUNK).astype(jnp.int32)

    mesh = plsc.VectorSubcoreMesh(core_axis_name="c", subcore_axis_name="s")
    out = pl.kernel(
        _gather_body,
        mesh=mesh,
        out_type=jax.ShapeDtypeStruct((B_TOTAL, D_MODEL), jnp.float32),
        scratch_types=[
            pltpu.VMEM((N_CHUNK, CHUNK), jnp.int32),
            pltpu.VMEM((K * CHUNK, D_MODEL), jnp.float32),
            pltpu.VMEM((K * CHUNK, D_MODEL), jnp.float32),
            pltpu.SemaphoreType.DMA,
            pltpu.SemaphoreType.DMA,
            pltpu.SemaphoreType.DMA,
            pltpu.SemaphoreType.DMA,
        ],
        compiler_params=pltpu.CompilerParams(use_tc_tiling_on_sc=False),
    )(idx, table)
    out = _repack(out.reshape(B_TOTAL // 2, 2 * D_MODEL))
    return out.reshape(BATCH, POS, D_MODEL)


def kernel(x, W_E):
    return _embed(x, W_E)


# speedup vs baseline: 1.3810x; 1.3810x over previous
import jax
import jax.numpy as jnp
from jax import lax
from jax.experimental import pallas as pl
from jax.experimental.pallas import tpu as pltpu
from jax.experimental.pallas import tpu_sc as plsc

D_MODEL = 64
D_VOCAB = 100000
BATCH = 4096
POS = 200

B_TOTAL = BATCH * POS
NW = 32
B_PER_W = B_TOTAL // NW
CHUNK = 128
N_CHUNK = B_PER_W // CHUNK

_TBLK = 4096


def _table_body(w_ref, out_ref):
    t = w_ref[...].T.reshape(_TBLK // 2, 2, D_MODEL)
    out_ref[...] = jnp.concatenate([t[:, 0, :], t[:, 1, :]], axis=1)


def _build_table(W_E):
    return pl.pallas_call(
        _table_body,
        grid=(pl.cdiv(D_VOCAB, _TBLK),),
        in_specs=[pl.BlockSpec((D_MODEL, _TBLK), lambda i: (0, i))],
        out_specs=pl.BlockSpec((_TBLK // 2, 2 * D_MODEL), lambda i: (i, 0)),
        out_shape=jax.ShapeDtypeStruct((D_VOCAB // 2, 2 * D_MODEL), jnp.float32),
    )(W_E)


_RBLK = 4096
_HALF = B_TOTAL // 2


def _repack_body(in_ref, out_ref):
    x = in_ref[...]
    out_ref[0] = x[:, :D_MODEL]
    out_ref[1] = x[:, D_MODEL:]


def _repack(packed):
    return pl.pallas_call(
        _repack_body,
        grid=(_HALF // _RBLK,),
        in_specs=[pl.BlockSpec((_RBLK, 2 * D_MODEL), lambda i: (i, 0))],
        out_specs=pl.BlockSpec((2, _RBLK, D_MODEL), lambda i: (0, i, 0)),
        out_shape=jax.ShapeDtypeStruct((2, _HALF, D_MODEL), jnp.float32),
    )(packed)


K = 4
NG = N_CHUNK // K
NC_H = N_CHUNK // 2
P_PER_W = (B_TOTAL // 2) // NW
G_ROWS = 2 * CHUNK


def _gather_body(idx_hbm, table_hbm, out_hbm, idx_v, buf_a, buf_b,
                 gsem_a, gsem_b, wsem_a, wsem_b):
    wid = lax.axis_index("s") * 2 + lax.axis_index("c")
    l_base = wid * NC_H
    r_base = 3200 + wid * NC_H
    out_base = wid * P_PER_W

    pltpu.sync_copy(idx_hbm.at[pl.ds(l_base, NC_H)], idx_v.at[pl.ds(0, NC_H)])
    pltpu.sync_copy(idx_hbm.at[pl.ds(r_base, NC_H)],
                    idx_v.at[pl.ds(NC_H, NC_H)])

    bufs = (buf_a, buf_b)
    gsems = (gsem_a, gsem_b)
    wsems = (wsem_a, wsem_b)

    def transfers(g, b):
        for j in range(K):
            half, sub = divmod(j, 2)
            yield (
                table_hbm.at[idx_v.at[half * NC_H + 2 * g + sub]],
                bufs[b].at[pl.ds(j * CHUNK, CHUNK)],
                gsems[b],
            )

    def fire(g, b):
        for src, dst, sem in transfers(g, b):
            pltpu.async_copy(src, dst, sem)

    def drain(g, b):
        for src, dst, sem in transfers(g, b):
            pltpu.make_async_copy(src, dst, sem).wait()

    def half_writes(g, b):
        for half in range(2):
            yield (
                bufs[b].at[pl.ds(half * G_ROWS, G_ROWS)],
                out_hbm.at[pl.ds(out_base + g * G_ROWS, G_ROWS),
                           pl.ds(half * D_MODEL, D_MODEL)],
                wsems[b],
            )

    def write(g, b):
        for src, dst, sem in half_writes(g, b):
            pltpu.async_copy(src, dst, sem)

    def wait_write(g, b):
        for src, dst, sem in half_writes(g, b):
            pltpu.make_async_copy(src, dst, sem).wait()

    fire(0, 0)

    def body(i, carry):
        g0 = 2 * i
        g1 = g0 + 1

        @pl.when(i > 0)
        def _():
            wait_write(g1 - 2, 1)
        fire(g1, 1)

        drain(g0, 0)
        write(g0, 0)

        @pl.when(i < NG // 2 - 1)
        def _():
            wait_write(g0, 0)
            fire(g0 + 2, 0)

        drain(g1, 1)
        write(g1, 1)
        return carry

    lax.fori_loop(0, NG // 2, body, 0)
    wait_write(NG - 2, 0)
    wait_write(NG - 1, 1)


@jax.jit
def _embed(x, W_E):
    table = _build_table(W_E).reshape(D_VOCAB, D_MODEL)
    idx = x.reshape(B_TOTAL // CHUNK, CHUNK).astype(jnp.int32)

    mesh = plsc.VectorSubcoreMesh(core_axis_name="c", subcore_axis_name="s")
    out = pl.kernel(
        _gather_body,
        mesh=mesh,
        out_type=jax.ShapeDtypeStruct((_HALF, 2 * D_MODEL), jnp.float32),
        scratch_types=[
            pltpu.VMEM((N_CHUNK, CHUNK), jnp.int32),
            pltpu.VMEM((K * CHUNK, D_MODEL), jnp.float32),
            pltpu.VMEM((K * CHUNK, D_MODEL), jnp.float32),
            pltpu.SemaphoreType.DMA,
            pltpu.SemaphoreType.DMA,
            pltpu.SemaphoreType.DMA,
            pltpu.SemaphoreType.DMA,
        ],
        compiler_params=pltpu.CompilerParams(use_tc_tiling_on_sc=False),
    )(idx, table)
    out = _repack(out)
    return out.reshape(BATCH, POS, D_MODEL)


def kernel(x, W_E):
    return _embed(x, W_E)
